# baseline (speedup 1.0000x reference)
import functools

import jax
import jax.numpy as jnp
from jax import lax
from jax.experimental import pallas as pl
from jax.experimental.pallas import tpu as pltpu
from jax.experimental.pallas import tpu_sc as plsc

B = 16384
DIM = 64
RANK = 8
HRANGE = 2 ** 14
OUT_BITS = 14
C0 = 2654435761
C1 = 2246822519
C2 = 3266489917

NC = 2
NS = 16
NW = NC * NS
EPW = B // NW
GRP = 16
NGRP = EPW // GRP
DB = 8
NDB = DIM // DB
RR = RANK * RANK


def _tt_body(x_hbm, s_hbm, e_hbm, m_hbm, out_hbm,
             xbuf, idx_m, mb, eb, sb, outb, idx_e, idx_s, sem):
    cid = lax.axis_index("c")
    sid = lax.axis_index("s")
    wid = cid * NS + sid
    base = wid * EPW
    pltpu.sync_copy(x_hbm.at[pl.ds(base, EPW)], xbuf)
    lanes = lax.iota(jnp.int32, GRP)

    def group_body(g, carry):
        xv = xbuf[pl.ds(g * GRP, GRP)]
        xu = xv.astype(jnp.uint32)
        sh = jnp.uint32(32 - OUT_BITS)
        h0 = lax.shift_right_logical(xu * jnp.uint32(C0), sh).astype(jnp.int32)
        h1 = lax.shift_right_logical(xu * jnp.uint32(C1), sh).astype(jnp.int32)
        h2 = lax.shift_right_logical(xu * jnp.uint32(C2), sh).astype(jnp.int32)
        idx_e[...] = h2
        idx_s[...] = h0
        h1d = h1 * DIM

        def fill(d, c):
            idx_m[pl.ds(d * GRP, GRP)] = h1d + d
            return c

        lax.fori_loop(0, DIM, fill, 0)
        pltpu.async_copy(e_hbm.at[idx_e], eb, sem).wait()
        pltpu.async_copy(s_hbm.at[idx_s], sb, sem).wait()

        def dblock(db, c):
            pltpu.async_copy(
                m_hbm.at[idx_m.at[pl.ds(db * (DB * GRP), DB * GRP)]], mb, sem
            ).wait()

            def dloop(dd, c2):
                d = db * DB + dd
                colbase = d * RANK
                rowv = dd * GRP + lanes
                sv = [plsc.load_gather(
                          sb, [lanes, jnp.full((GRP,), colbase + s, jnp.int32)])
                      for s in range(RANK)]
                vv = [plsc.load_gather(
                          eb, [lanes, jnp.full((GRP,), colbase + r, jnp.int32)])
                      for r in range(RANK)]
                acc = jnp.zeros((GRP,), jnp.float32)
                for r in range(RANK):
                    t = jnp.zeros((GRP,), jnp.float32)
                    for s in range(RANK):
                        m = plsc.load_gather(
                            mb, [rowv, jnp.full((GRP,), r * RANK + s, jnp.int32)])
                        t = t + m * sv[s]
                    acc = acc + t * vv[r]
                plsc.store_scatter(
                    outb, [lanes, jnp.full((GRP,), d, jnp.int32)], acc)
                return c2

            lax.fori_loop(0, DB, dloop, 0)
            return c

        lax.fori_loop(0, NDB, dblock, 0)
        pltpu.sync_copy(outb, out_hbm.at[pl.ds(base + g * GRP, GRP), :])
        return carry

    lax.fori_loop(0, NGRP, group_body, 0)


@jax.jit
def _tt_embed(x, s2, e2, m2):
    mesh = plsc.VectorSubcoreMesh(core_axis_name="c", subcore_axis_name="s")
    f = functools.partial(
        pl.kernel,
        out_type=jax.ShapeDtypeStruct((B, DIM), jnp.float32),
        mesh=mesh,
        scratch_types=[
            pltpu.VMEM((EPW,), jnp.int32),
            pltpu.VMEM((DIM * GRP,), jnp.int32),
            pltpu.VMEM((DB * GRP, RR), jnp.float32),
            pltpu.VMEM((GRP, DIM * RANK), jnp.float32),
            pltpu.VMEM((GRP, DIM * RANK), jnp.float32),
            pltpu.VMEM((GRP, DIM), jnp.float32),
            pltpu.VMEM((GRP,), jnp.int32),
            pltpu.VMEM((GRP,), jnp.int32),
            pltpu.SemaphoreType.DMA,
        ],
        compiler_params=pltpu.CompilerParams(
            use_tc_tiling_on_sc=False, needs_layout_passes=False),
    )(_tt_body)
    return f(x, s2, e2, m2)


def kernel(x, start_core, end_core, cores):
    s2 = start_core.reshape(HRANGE, DIM * RANK)
    e2 = end_core.reshape(HRANGE, DIM * RANK)
    m2 = cores[0].reshape(HRANGE * DIM, RR)
    return _tt_embed(x, s2, e2, m2)
